# Initial kernel scaffold; baseline (speedup 1.0000x reference)
import jax
import jax.numpy as jnp
from jax.experimental import pallas as pl


def kernel(queries, keys, values, attn_mask, index_sample):
    raise NotImplementedError("write your pallas kernel here")


# speedup vs baseline: 1.8581x; 1.8581x over previous
import functools
from math import sqrt

import jax
import jax.numpy as jnp
import numpy as np
from jax import lax
from jax.experimental import pallas as pl
from jax.experimental.pallas import tpu as pltpu
from jax.experimental.pallas import tpu_sc as plsc


def _make_sc_m_scores(BH, L, D, S):
    info = plsc.get_sparse_core_info()
    NC, NS = info.num_cores, info.num_subcores
    NW = NC * NS
    ROWS = BH * L
    per_w = ROWS // NW
    G = 8
    iters = per_w // G
    assert per_w * NW == ROWS and iters * G == per_w and iters % 2 == 0
    mesh = plsc.VectorSubcoreMesh(core_axis_name="c", subcore_axis_name="s")
    inv_lk = 1.0 / float(L)

    @functools.partial(
        pl.kernel,
        out_type=jax.ShapeDtypeStruct((ROWS,), jnp.float32),
        mesh=mesh,
        scratch_types=[
            pltpu.VMEM((G * S,), jnp.int32),
            pltpu.VMEM((G * S,), jnp.int32),
            pltpu.VMEM((G * D,), jnp.float32),
            pltpu.VMEM((G * D,), jnp.float32),
            pltpu.VMEM((G * S, D), jnp.float32),
            pltpu.VMEM((G * S, D), jnp.float32),
            pltpu.VMEM((per_w,), jnp.float32),
            pltpu.SemaphoreType.DMA,
            pltpu.SemaphoreType.DMA,
        ],
        compiler_params=pltpu.CompilerParams(
            needs_layout_passes=False, use_tc_tiling_on_sc=False
        ),
    )
    def sc_m(q_hbm, k_hbm, idx_hbm, m_hbm,
             idx0, idx1, q0, q1, kg0, kg1, m_v, sem0, sem1):
        wid = lax.axis_index("s") * NC + lax.axis_index("c")
        row0 = wid * per_w
        banks = ((idx0, q0, kg0, sem0), (idx1, q1, kg1, sem1))

        def issue(i, b):
            idx_v, q_v, kg_v, sem = banks[b]
            base = row0 + i * G
            pltpu.sync_copy(idx_hbm.at[pl.ds(base * S, G * S)], idx_v)
            pltpu.sync_copy(q_hbm.at[pl.ds(base * D, G * D)], q_v)
            for g in range(G):
                pltpu.async_copy(
                    k_hbm.at[idx_v.at[pl.ds(g * S, S)]],
                    kg_v.at[pl.ds(g * S, S)],
                    sem,
                )

        def drain(b):
            _, _, kg_v, sem = banks[b]
            pltpu.make_async_copy(k_hbm.at[pl.ds(0, G * S)], kg_v, sem).wait()

        lane = lax.iota(jnp.int32, 16)

        def compute(b, lane_base, mvec):
            _, q_v, kg_v, _ = banks[b]
            for g in range(G):
                q = [q_v[pl.ds(g * D + j * 16, 16)] for j in range(D // 16)]
                smax = None
                ssum = None
                for s in range(S):
                    row = jnp.full((16,), g * S + s, jnp.int32)
                    prod = None
                    for j in range(D // 16):
                        kv = plsc.load_gather(kg_v, [row, lane + j * 16])
                        term = q[j] * kv
                        prod = term if prod is None else prod + term
                    dval = jnp.sum(prod)
                    smax = dval if smax is None else jnp.maximum(smax, dval)
                    ssum = dval if ssum is None else ssum + dval
                mval = smax - ssum * inv_lk
                mvec = jnp.where(lane == (lane_base + g), mval, mvec)
            return mvec

        issue(0, 0)

        def body(i2, _):
            i = i2 * 2
            mvec = jnp.zeros((16,), jnp.float32)
            drain(0)

            @pl.when(i + 1 < iters)
            def _():
                issue(i + 1, 1)

            mvec = compute(0, 0, mvec)
            drain(1)

            @pl.when(i + 2 < iters)
            def _():
                issue(i + 2, 0)

            mvec = compute(1, G, mvec)
            m_v[pl.ds(i2 * 2 * G, 2 * G)] = mvec
            return 0

        lax.fori_loop(0, iters // 2, body, 0)
        pltpu.sync_copy(m_v, m_hbm.at[pl.ds(row0, per_w)])

    return sc_m


def _tc_attn_body(m_ref, q_ref, k_ref, v_ref, o_ref, *, U, L, D, scale):
    m = m_ref[0]
    iota = lax.broadcasted_iota(jnp.int32, (1, L), 1)
    q2 = q_ref[0]
    k2 = k_ref[0]
    v2 = v_ref[0]

    idxs = []
    rows = []
    work = m
    for _ in range(U):
        cur = jnp.max(work)
        sel = jnp.min(jnp.where(work == cur, iota, jnp.int32(L)))
        idxs.append(sel)
        rows.append(q_ref[0, pl.ds(sel, 1), :])
        work = jnp.where(iota == sel, -jnp.inf, work)

    qr = jnp.concatenate(rows, axis=0)
    scores = lax.dot_general(
        qr, k2, (((1,), (1,)), ((), ())),
        preferred_element_type=jnp.float32,
    ) * scale
    mx = jnp.max(scores, axis=1, keepdims=True)
    e = jnp.exp(scores - mx)
    attn = e / jnp.sum(e, axis=1, keepdims=True)
    upd = lax.dot_general(
        attn, v2, (((1,), (0,)), ((), ())),
        preferred_element_type=jnp.float32,
    )

    vmean = jnp.mean(v2, axis=0, keepdims=True)
    o_ref[0] = jnp.broadcast_to(vmean, (L, D))
    for i, sel in enumerate(idxs):
        o_ref[0, pl.ds(sel, 1), :] = upd[i : i + 1, :]


def kernel(queries, keys, values, attn_mask, index_sample):
    B, L, H, D = queries.shape
    S = index_sample.shape[1]
    BH = B * H
    u = min(5 * int(np.ceil(np.log(L))), L)
    scale = 1.0 / sqrt(D)

    qt = jnp.transpose(queries, (0, 2, 1, 3)).reshape(BH, L, D)
    kt = jnp.transpose(keys, (0, 2, 1, 3)).reshape(BH, L, D)
    vt = jnp.transpose(values, (0, 2, 1, 3)).reshape(BH, L, D)
    q_flat = qt.reshape(BH * L, D)
    k_flat = kt.reshape(BH * L, D)
    offs = (jnp.arange(BH, dtype=jnp.int32) * L)[:, None, None]
    idxo = (index_sample.astype(jnp.int32)[None, :, :] + offs).reshape(-1)

    def _round_bf16(x):
        xi = lax.bitcast_convert_type(x, jnp.int32)
        xi = xi + jnp.int32(0x7FFF) + ((xi >> 16) & 1)
        xi = jnp.bitwise_and(xi, jnp.int32(-65536))
        return lax.bitcast_convert_type(xi, jnp.float32)

    q_m = _round_bf16(q_flat)
    k_m = _round_bf16(k_flat)
    m_flat = _make_sc_m_scores(BH, L, D, S)(q_m.reshape(-1), k_m, idxo)
    m2 = m_flat.reshape(BH, 1, L)

    out = pl.pallas_call(
        functools.partial(_tc_attn_body, U=u, L=L, D=D, scale=scale),
        grid=(BH,),
        in_specs=[
            pl.BlockSpec((1, 1, L), lambda i: (i, 0, 0)),
            pl.BlockSpec((1, L, D), lambda i: (i, 0, 0)),
            pl.BlockSpec((1, L, D), lambda i: (i, 0, 0)),
            pl.BlockSpec((1, L, D), lambda i: (i, 0, 0)),
        ],
        out_specs=pl.BlockSpec((1, L, D), lambda i: (i, 0, 0)),
        out_shape=jax.ShapeDtypeStruct((BH, L, D), jnp.float32),
    )(m2, qt, kt, vt)
    return jnp.transpose(out.reshape(B, H, L, D), (0, 2, 1, 3))
